# baseline (speedup 1.0000x reference)
import jax
import jax.numpy as jnp
from jax.experimental import pallas as pl
from jax.experimental.pallas import tpu as pltpu

_TID = 288
_DIW = 7
_BB = 32
_DEPTH = 4
_CORES = 2


def _st_kernel(xt_ref, idx_ref, w1t_ref, wtt_ref, wdt_ref, bt_ref, o_ref,
               scratch, sems):
    n = xt_ref.shape[2]
    c = pl.program_id(0)
    k = pl.program_id(1)
    n_local = pl.num_programs(1)
    step = c * n_local + k
    slot = jax.lax.rem(k, _DEPTH)

    @pl.when(k >= _DEPTH)
    def _wait_slot():
        pltpu.make_async_copy(scratch.at[slot], scratch.at[slot],
                              sems.at[slot]).wait()

    row_t = jax.lax.broadcasted_iota(jnp.int32, (_TID, n), 0)
    row_d = jax.lax.broadcasted_iota(jnp.int32, (8, n), 0)
    bias = bt_ref[...]
    for j in range(_BB):
        f = xt_ref[j]
        tid = idx_ref[j, 0]
        diw = idx_ref[j, 1]
        oh_t = (row_t == tid[None, :]).astype(jnp.bfloat16)
        oh_d = (row_d == diw[None, :]).astype(jnp.bfloat16)
        acc = jnp.dot(w1t_ref[...], f, preferred_element_type=jnp.float32)
        acc = acc + jnp.dot(wtt_ref[...], oh_t, preferred_element_type=jnp.float32)
        acc = acc + jnp.dot(wdt_ref[...], oh_d, preferred_element_type=jnp.float32)
        scratch[slot, j] = acc + bias

    pltpu.make_async_copy(
        scratch.at[slot],
        o_ref.at[pl.ds(step * _BB, _BB), :, 0, :],
        sems.at[slot],
    ).start()

    @pl.when(k == n_local - 1)
    def _drain():
        for d in range(_DEPTH):
            pltpu.make_async_copy(scratch.at[d], scratch.at[d],
                                  sems.at[d]).wait()


def kernel(x, w_conv, w_tab, bias_node):
    B, L, N, C = x.shape
    K, Eo = w_conv.shape

    xt = jnp.transpose(x, (0, 1, 3, 2)).reshape(B, K, N).astype(jnp.bfloat16)
    tid = jnp.clip((x[:, -1, :, 1] * 288.0).astype(jnp.int32), 0, _TID - 1)
    diw = jnp.clip(x[:, -1, :, 2].astype(jnp.int32), 0, _DIW - 1)
    idx = jnp.stack([tid, diw], axis=1)

    w1t = w_conv.T.astype(jnp.bfloat16)
    wtt = w_tab[:_TID].T.astype(jnp.bfloat16)
    wdt = w_tab[_TID:_TID + 8].T.astype(jnp.bfloat16)
    biast = bias_node.T

    steps = B // _BB
    local = steps // _CORES
    out = pl.pallas_call(
        _st_kernel,
        out_shape=jax.ShapeDtypeStruct((B, Eo, 1, N), jnp.float32),
        grid=(_CORES, local),
        in_specs=[
            pl.BlockSpec((_BB, K, N), lambda c, k: (c * (B // _BB // _CORES) + k, 0, 0)),
            pl.BlockSpec((_BB, 2, N), lambda c, k: (c * (B // _BB // _CORES) + k, 0, 0)),
            pl.BlockSpec((Eo, K), lambda c, k: (0, 0)),
            pl.BlockSpec((Eo, _TID), lambda c, k: (0, 0)),
            pl.BlockSpec((Eo, 8), lambda c, k: (0, 0)),
            pl.BlockSpec((Eo, N), lambda c, k: (0, 0)),
        ],
        out_specs=pl.BlockSpec(memory_space=pl.ANY),
        scratch_shapes=[
            pltpu.VMEM((_DEPTH, _BB, Eo, N), jnp.float32),
            pltpu.SemaphoreType.DMA((_DEPTH,)),
        ],
        compiler_params=pltpu.CompilerParams(
            dimension_semantics=("parallel", "arbitrary")),
    )(xt, idx, w1t, wtt, wdt, biast)

    return jnp.transpose(out, (0, 1, 3, 2))


# speedup vs baseline: 1.0168x; 1.0168x over previous
import jax
import jax.numpy as jnp
from jax.experimental import pallas as pl
from jax.experimental.pallas import tpu as pltpu

_TID = 288
_DIW = 7
_BB = 16
_DEPTH = 4
_CORES = 2


def _st_kernel(xt_ref, idx_ref, w1t_ref, wtt_ref, wdt_ref, bt_ref, o_ref,
               scratch, sems):
    n = xt_ref.shape[2]
    c = pl.program_id(0)
    k = pl.program_id(1)
    n_local = pl.num_programs(1)
    step = c * n_local + k
    slot = jax.lax.rem(k, _DEPTH)

    @pl.when(k >= _DEPTH)
    def _wait_slot():
        pltpu.make_async_copy(scratch.at[slot], scratch.at[slot],
                              sems.at[slot]).wait()

    row_t = jax.lax.broadcasted_iota(jnp.int32, (_TID, n), 0)
    row_d = jax.lax.broadcasted_iota(jnp.int32, (8, n), 0)
    bias = bt_ref[...]
    for j in range(_BB):
        f = xt_ref[j]
        tid = idx_ref[j, 0]
        diw = idx_ref[j, 1]
        oh_t = (row_t == tid[None, :]).astype(jnp.bfloat16)
        oh_d = (row_d == diw[None, :]).astype(jnp.bfloat16)
        acc = jnp.dot(w1t_ref[...], f, preferred_element_type=jnp.float32)
        acc = acc + jnp.dot(wtt_ref[...], oh_t, preferred_element_type=jnp.float32)
        acc = acc + jnp.dot(wdt_ref[...], oh_d, preferred_element_type=jnp.float32)
        scratch[slot, j] = acc + bias

    pltpu.make_async_copy(
        scratch.at[slot],
        o_ref.at[pl.ds(step * _BB, _BB), :, 0, :],
        sems.at[slot],
    ).start()

    @pl.when(k == n_local - 1)
    def _drain():
        for d in range(_DEPTH):
            pltpu.make_async_copy(scratch.at[d], scratch.at[d],
                                  sems.at[d]).wait()


def kernel(x, w_conv, w_tab, bias_node):
    B, L, N, C = x.shape
    K, Eo = w_conv.shape

    xt = jnp.transpose(x, (0, 1, 3, 2)).reshape(B, K, N).astype(jnp.bfloat16)
    tid = jnp.clip((x[:, -1, :, 1] * 288.0).astype(jnp.int32), 0, _TID - 1)
    diw = jnp.clip(x[:, -1, :, 2].astype(jnp.int32), 0, _DIW - 1)
    idx = jnp.stack([tid, diw], axis=1)

    w1t = w_conv.T.astype(jnp.bfloat16)
    wtt = w_tab[:_TID].T.astype(jnp.bfloat16)
    wdt = w_tab[_TID:_TID + 8].T.astype(jnp.bfloat16)
    biast = bias_node.T

    steps = B // _BB
    local = steps // _CORES
    out = pl.pallas_call(
        _st_kernel,
        out_shape=jax.ShapeDtypeStruct((B, Eo, 1, N), jnp.float32),
        grid=(_CORES, local),
        in_specs=[
            pl.BlockSpec((_BB, K, N), lambda c, k: (c * (B // _BB // _CORES) + k, 0, 0)),
            pl.BlockSpec((_BB, 2, N), lambda c, k: (c * (B // _BB // _CORES) + k, 0, 0)),
            pl.BlockSpec((Eo, K), lambda c, k: (0, 0)),
            pl.BlockSpec((Eo, _TID), lambda c, k: (0, 0)),
            pl.BlockSpec((Eo, 8), lambda c, k: (0, 0)),
            pl.BlockSpec((Eo, N), lambda c, k: (0, 0)),
        ],
        out_specs=pl.BlockSpec(memory_space=pl.ANY),
        scratch_shapes=[
            pltpu.VMEM((_DEPTH, _BB, Eo, N), jnp.float32),
            pltpu.SemaphoreType.DMA((_DEPTH,)),
        ],
        compiler_params=pltpu.CompilerParams(
            dimension_semantics=("parallel", "arbitrary")),
    )(xt, idx, w1t, wtt, wdt, biast)

    return jnp.transpose(out, (0, 1, 3, 2))
